# baseline (speedup 1.0000x reference)
import functools

import jax
import jax.numpy as jnp
from jax import lax
from jax.experimental import pallas as pl
from jax.experimental.pallas import tpu as pltpu
from jax.experimental.pallas import tpu_sc as plsc


def _red6(x):
    c = [x[:, i * 128:(i + 1) * 128] for i in range(6)]
    r01 = jnp.sum(c[0] + c[1], axis=1, keepdims=True)
    r23 = jnp.sum(c[2] + c[3], axis=1, keepdims=True)
    r45 = jnp.sum(c[4] + c[5], axis=1, keepdims=True)
    return (r01 + r23) + r45


def _sim_body(feat_ref, hup_ref, hdn_ref, out_ref, featT_ref):
    t = pl.program_id(1)
    ft = jnp.transpose(feat_ref[0], (1, 0))
    hup = jnp.transpose(hup_ref[0][:, 64:], (1, 0))
    hdn = jnp.transpose(hdn_ref[0][:, :64], (1, 0))
    featT_ref[0] = ft
    i = lax.broadcasted_iota(jnp.int32, (512, 1), 0)
    x = jnp.bitwise_and(i, 63)
    y = t * 8 + jnp.right_shift(i, 6)
    eps = jnp.float32(1e-4)

    down = jnp.concatenate([hup[-1:], ft[:-1]], axis=0)
    up = jnp.concatenate([ft[1:], hdn[:1]], axis=0)
    down64 = jnp.concatenate([hup, ft[:-64]], axis=0)
    up64 = jnp.concatenate([ft[64:], hdn], axis=0)

    n_t = _red6(ft * ft)
    n_hup = _red6(hup * hup)
    n_hdn = _red6(hdn * hdn)
    n_full = jnp.concatenate([n_hup, n_t, n_hdn], axis=0)
    sq_full = jnp.sqrt(n_full)
    sq_dn = sq_full[63:575]
    sq_up = sq_full[65:577]
    sq_dn64 = sq_full[0:512]
    sq_up64 = sq_full[128:640]

    num_h = _red6(down * up)
    num_v = _red6(down64 * up64)
    h = num_h / (sq_dn * sq_up)
    v = num_v / (sq_dn64 * sq_up64)

    f3 = ft.reshape(8, 64, 768)
    r0 = f3[:, 0, :]
    r1 = f3[:, 1, :]
    r62 = f3[:, 62, :]
    r63 = f3[:, 63, :]
    num_x0 = _red6((r0 * r1) * eps)
    e0 = eps * r0
    d1_x0 = _red6(e0 * e0)
    num_x63 = _red6((r62 * r63) * eps)
    e63 = eps * r63
    d2_x63 = _red6(e63 * e63)
    sqn3 = sq_full[64:576].reshape(8, 64, 1)
    h_x0 = num_x0 / (jnp.sqrt(d1_x0) * sqn3[:, 1, :])
    h_x63 = num_x63 / (sqn3[:, 62, :] * jnp.sqrt(d2_x63))
    h3 = h.reshape(8, 64, 1)
    h3 = jnp.where(lax.broadcasted_iota(jnp.int32, (8, 64, 1), 1) == 0,
                   h_x0[:, None, :], h3)
    h3 = jnp.where(lax.broadcasted_iota(jnp.int32, (8, 64, 1), 1) == 63,
                   h_x63[:, None, :], h3)
    h = h3.reshape(512, 1)

    s0 = ft[:64]
    s0n = up64[:64]
    num_y0 = _red6((s0n * s0) * eps)
    ey0 = eps * s0
    d2_y0 = _red6(ey0 * ey0)
    v_y0 = num_y0 / (sq_up64[:64] * jnp.sqrt(d2_y0))
    s63 = ft[448:]
    s63p = down64[448:]
    num_y63 = _red6((s63p * s63) * eps)
    ey63 = eps * s63
    d1_y63 = _red6(ey63 * ey63)
    v_y63 = num_y63 / (jnp.sqrt(d1_y63) * sq_dn64[448:])
    zfill = jnp.zeros((448, 1), jnp.float32)
    v = jnp.where(y == 0, jnp.concatenate([v_y0, zfill], axis=0), v)
    v = jnp.where(y == 63, jnp.concatenate([zfill, v_y63], axis=0), v)

    out_ref[0] = (h + v) * jnp.float32(0.5)


def _sim_map(featO):
    B, C, HW = featO.shape
    sim, featT = pl.pallas_call(
        _sim_body,
        grid=(B, 8),
        in_specs=[
            pl.BlockSpec((1, C, 512), lambda b, t: (b, 0, t)),
            pl.BlockSpec((1, C, 128), lambda b, t: (b, 0, jnp.maximum(4 * t - 1, 0))),
            pl.BlockSpec((1, C, 128), lambda b, t: (b, 0, jnp.minimum(4 * t + 4, 31))),
        ],
        out_specs=[
            pl.BlockSpec((1, 512, 1), lambda b, t: (b, t, 0)),
            pl.BlockSpec((1, 512, C), lambda b, t: (b, t, 0)),
        ],
        out_shape=[
            jax.ShapeDtypeStruct((B, HW, 1), jnp.float32),
            jax.ShapeDtypeStruct((B, HW, C), jnp.float32),
        ],
    )(featO, featO, featO)
    return sim[..., 0], featT


def _topk_body(sim_ref, koff_ref, vals_ref, px_ref, py_ref, ginds_ref):
    sim = sim_ref[...]
    B, N = sim.shape
    lane = lax.broadcasted_iota(jnp.int32, (B, N), 1)
    xcol = jnp.bitwise_and(lane, 63)
    inf = jnp.float32(jnp.inf)
    one = jnp.float32(1.0)

    sl = jnp.concatenate([sim[:, 1:], sim[:, :1]], axis=1)
    sr = jnp.concatenate([sim[:, -1:], sim[:, :-1]], axis=1)
    nb_r = jnp.where(xcol == 63, inf, sl)
    nb_l = jnp.where(xcol == 0, inf, sr)
    min_h = jnp.minimum(jnp.minimum(nb_l, sim), nb_r)
    su = jnp.concatenate([sim[:, 64:], sim[:, :64]], axis=1)
    sd = jnp.concatenate([sim[:, -64:], sim[:, :-64]], axis=1)
    nb_d = jnp.where(lane >= N - 64, inf, su)
    nb_u = jnp.where(lane < 64, inf, sd)
    min_v = jnp.minimum(jnp.minimum(nb_u, sim), nb_d)

    minima = (sim == min_h) | (sim == min_v)
    sel = jnp.where(minima, one - sim, jnp.float32(0.0))
    score = one - (one - sel)

    val = score
    idx = lane
    for kk in [2 ** e for e in range(1, 13)]:
        j = kk // 2
        while j >= 1:
            rl = jnp.concatenate([val[:, j:], val[:, :j]], axis=1)
            rr = jnp.concatenate([val[:, -j:], val[:, :-j]], axis=1)
            il = jnp.concatenate([idx[:, j:], idx[:, :j]], axis=1)
            ir = jnp.concatenate([idx[:, -j:], idx[:, :-j]], axis=1)
            is_low = (lane & j) == 0
            ov = jnp.where(is_low, rl, rr)
            oi = jnp.where(is_low, il, ir)
            desc = (lane & kk) == 0
            self_greater = (val > ov) | ((val == ov) & (idx < oi))
            keep = self_greater == (desc == is_low)
            val = jnp.where(keep, val, ov)
            idx = jnp.where(keep, idx, oi)
            j //= 2

    koff = koff_ref[0, 0]
    vals_ref[...] = val[:, :1024]
    inds = idx[:, :1024] + koff
    py_ref[...] = jnp.floor_divide(inds, 64)
    px_ref[...] = jnp.remainder(inds, 64)
    gi = jnp.where(inds < 0, inds + 4096, inds)
    gi = jnp.clip(gi, 0, 4095)
    boff = lax.broadcasted_iota(jnp.int32, gi.shape, 0) * 4096
    ginds_ref[...] = gi + boff


def _nms_topk(sim, koff, interpret=False):
    B, N = sim.shape
    return pl.pallas_call(
        _topk_body,
        in_specs=[
            pl.BlockSpec((B, N), lambda: (0, 0)),
            pl.BlockSpec(memory_space=pltpu.SMEM),
        ],
        out_shape=(
            jax.ShapeDtypeStruct((B, 1024), jnp.float32),
            jax.ShapeDtypeStruct((B, 1024), jnp.int32),
            jax.ShapeDtypeStruct((B, 1024), jnp.int32),
            jax.ShapeDtypeStruct((B, 1024), jnp.int32),
        ),
        interpret=interpret,
    )(sim, koff)


def _gather_body(featT_hbm, ginds_hbm, out_hbm, idx_v, rows_v, sem):
    nc = 2
    wid = lax.axis_index("s") * nc + lax.axis_index("c")
    for ci in range(16):
        g0 = wid * 512 + ci * 32
        pltpu.sync_copy(ginds_hbm.at[pl.ds(g0, 32)], idx_v)
        pltpu.async_copy(featT_hbm.at[idx_v], rows_v, sem).wait()
        pltpu.sync_copy(rows_v, out_hbm.at[pl.ds(g0, 32)])


def _sc_gather(featT, ginds):
    B, HW, C = featT.shape
    mesh = plsc.VectorSubcoreMesh(core_axis_name="c", subcore_axis_name="s")
    kern = functools.partial(
        pl.kernel,
        out_type=jax.ShapeDtypeStruct((B * 1024, C), jnp.float32),
        scratch_types=[
            pltpu.VMEM((32,), jnp.int32),
            pltpu.VMEM((32, C), jnp.float32),
            pltpu.SemaphoreType.DMA,
        ],
        mesh=mesh,
    )(_gather_body)
    out = kern(featT.reshape(B * HW, C), ginds.reshape(B * 1024))
    return out.reshape(B, 1024, C)


def kernel(features, k):
    B, C, H, W = features.shape
    sim, featT = _sim_map(features.reshape(B, C, H * W))
    koff = (jnp.asarray(k, jnp.int32) - 1024).reshape(1, 1)
    vals, px, py, ginds = _nms_topk(sim, koff)
    pos = jnp.stack([px, py], axis=-1)
    point_feats = _sc_gather(featT, ginds)
    return point_feats, pos, vals


# speedup vs baseline: 2.0824x; 2.0824x over previous
import functools

import jax
import jax.numpy as jnp
from jax import lax
from jax.experimental import pallas as pl
from jax.experimental.pallas import tpu as pltpu
from jax.experimental.pallas import tpu_sc as plsc


def _red6(x):
    c = [x[:, i * 128:(i + 1) * 128] for i in range(6)]
    r01 = jnp.sum(c[0] + c[1], axis=1, keepdims=True)
    r23 = jnp.sum(c[2] + c[3], axis=1, keepdims=True)
    r45 = jnp.sum(c[4] + c[5], axis=1, keepdims=True)
    return (r01 + r23) + r45


def _sim_body(feat_ref, hup_ref, hdn_ref, out_ref):
    t = pl.program_id(1)
    ft = feat_ref[0]
    hup = hup_ref[0]
    hdn = hdn_ref[0]
    i = lax.broadcasted_iota(jnp.int32, (1024, 1), 0)
    x = jnp.bitwise_and(i, 63)
    y = t * 16 + jnp.right_shift(i, 6)
    eps = jnp.float32(1e-4)

    down = jnp.concatenate([hup[-1:], ft[:-1]], axis=0)
    up = jnp.concatenate([ft[1:], hdn[:1]], axis=0)
    down64 = jnp.concatenate([hup, ft[:-64]], axis=0)
    up64 = jnp.concatenate([ft[64:], hdn], axis=0)

    n_t = _red6(ft * ft)
    n_hup = _red6(hup * hup)
    n_hdn = _red6(hdn * hdn)
    n_full = jnp.concatenate([n_hup, n_t, n_hdn], axis=0)
    sq_full = jnp.sqrt(n_full)
    sq_dn = sq_full[63:1087]
    sq_up = sq_full[65:1089]
    sq_dn64 = sq_full[0:1024]
    sq_up64 = sq_full[128:1152]

    num_h = _red6(down * up)
    num_v = _red6(down64 * up64)
    h = num_h / (sq_dn * sq_up)
    v = num_v / (sq_dn64 * sq_up64)

    f3 = ft.reshape(16, 64, 768)
    r0 = f3[:, 0, :]
    r1 = f3[:, 1, :]
    r62 = f3[:, 62, :]
    r63 = f3[:, 63, :]
    num_x0 = _red6((r0 * r1) * eps)
    e0 = eps * r0
    d1_x0 = _red6(e0 * e0)
    num_x63 = _red6((r62 * r63) * eps)
    e63 = eps * r63
    d2_x63 = _red6(e63 * e63)
    sqn3 = sq_full[64:1088].reshape(16, 64, 1)
    h_x0 = num_x0 / (jnp.sqrt(d1_x0) * sqn3[:, 1, :])
    h_x63 = num_x63 / (sqn3[:, 62, :] * jnp.sqrt(d2_x63))
    h3 = h.reshape(16, 64, 1)
    h3 = jnp.where(lax.broadcasted_iota(jnp.int32, (16, 64, 1), 1) == 0,
                   h_x0[:, None, :], h3)
    h3 = jnp.where(lax.broadcasted_iota(jnp.int32, (16, 64, 1), 1) == 63,
                   h_x63[:, None, :], h3)
    h = h3.reshape(1024, 1)

    s0 = ft[:64]
    s0n = up64[:64]
    num_y0 = _red6((s0n * s0) * eps)
    ey0 = eps * s0
    d2_y0 = _red6(ey0 * ey0)
    v_y0 = num_y0 / (sq_up64[:64] * jnp.sqrt(d2_y0))
    s63 = ft[960:]
    s63p = down64[960:]
    num_y63 = _red6((s63p * s63) * eps)
    ey63 = eps * s63
    d1_y63 = _red6(ey63 * ey63)
    v_y63 = num_y63 / (jnp.sqrt(d1_y63) * sq_dn64[960:])
    zfill = jnp.zeros((960, 1), jnp.float32)
    v = jnp.where(y == 0, jnp.concatenate([v_y0, zfill], axis=0), v)
    v = jnp.where(y == 63, jnp.concatenate([zfill, v_y63], axis=0), v)

    out_ref[0] = (h + v) * jnp.float32(0.5)


def _sim_map(featT):
    B, HW, C = featT.shape
    out = pl.pallas_call(
        _sim_body,
        grid=(B, 4),
        in_specs=[
            pl.BlockSpec((1, 1024, C), lambda b, t: (b, t, 0)),
            pl.BlockSpec((1, 64, C), lambda b, t: (b, jnp.maximum(16 * t - 1, 0), 0)),
            pl.BlockSpec((1, 64, C), lambda b, t: (b, jnp.minimum(16 * t + 16, 63), 0)),
        ],
        out_specs=pl.BlockSpec((1, 1024, 1), lambda b, t: (b, t, 0)),
        out_shape=jax.ShapeDtypeStruct((B, HW, 1), jnp.float32),
    )(featT, featT, featT)
    return out[..., 0]


def _topk_body(sim_ref, koff_ref, vals_ref, px_ref, py_ref, ginds_ref):
    sim = sim_ref[...]
    B, N = sim.shape
    lane = lax.broadcasted_iota(jnp.int32, (B, N), 1)
    xcol = jnp.bitwise_and(lane, 63)
    inf = jnp.float32(jnp.inf)
    one = jnp.float32(1.0)

    sl = jnp.concatenate([sim[:, 1:], sim[:, :1]], axis=1)
    sr = jnp.concatenate([sim[:, -1:], sim[:, :-1]], axis=1)
    nb_r = jnp.where(xcol == 63, inf, sl)
    nb_l = jnp.where(xcol == 0, inf, sr)
    min_h = jnp.minimum(jnp.minimum(nb_l, sim), nb_r)
    su = jnp.concatenate([sim[:, 64:], sim[:, :64]], axis=1)
    sd = jnp.concatenate([sim[:, -64:], sim[:, :-64]], axis=1)
    nb_d = jnp.where(lane >= N - 64, inf, su)
    nb_u = jnp.where(lane < 64, inf, sd)
    min_v = jnp.minimum(jnp.minimum(nb_u, sim), nb_d)

    minima = (sim == min_h) | (sim == min_v)
    sel = jnp.where(minima, one - sim, jnp.float32(0.0))
    score = one - (one - sel)

    val = score
    idx = lane
    for kk in [2 ** e for e in range(1, 13)]:
        j = kk // 2
        while j >= 1:
            rl = jnp.concatenate([val[:, j:], val[:, :j]], axis=1)
            rr = jnp.concatenate([val[:, -j:], val[:, :-j]], axis=1)
            il = jnp.concatenate([idx[:, j:], idx[:, :j]], axis=1)
            ir = jnp.concatenate([idx[:, -j:], idx[:, :-j]], axis=1)
            is_low = (lane & j) == 0
            ov = jnp.where(is_low, rl, rr)
            oi = jnp.where(is_low, il, ir)
            desc = (lane & kk) == 0
            self_greater = (val > ov) | ((val == ov) & (idx < oi))
            keep = self_greater == (desc == is_low)
            val = jnp.where(keep, val, ov)
            idx = jnp.where(keep, idx, oi)
            j //= 2

    koff = koff_ref[0, 0]
    vals_ref[...] = val[:, :1024]
    inds = idx[:, :1024] + koff
    py_ref[...] = jnp.floor_divide(inds, 64)
    px_ref[...] = jnp.remainder(inds, 64)
    gi = jnp.where(inds < 0, inds + 4096, inds)
    gi = jnp.clip(gi, 0, 4095)
    boff = lax.broadcasted_iota(jnp.int32, gi.shape, 0) * 4096
    ginds_ref[...] = gi + boff


def _nms_topk(sim, koff, interpret=False):
    B, N = sim.shape
    return pl.pallas_call(
        _topk_body,
        in_specs=[
            pl.BlockSpec((B, N), lambda: (0, 0)),
            pl.BlockSpec(memory_space=pltpu.SMEM),
        ],
        out_shape=(
            jax.ShapeDtypeStruct((B, 1024), jnp.float32),
            jax.ShapeDtypeStruct((B, 1024), jnp.int32),
            jax.ShapeDtypeStruct((B, 1024), jnp.int32),
            jax.ShapeDtypeStruct((B, 1024), jnp.int32),
        ),
        interpret=interpret,
    )(sim, koff)


def _gather_body(featT_hbm, ginds_hbm, out_hbm, idx_v, rows_v, sem):
    nc = 2
    wid = lax.axis_index("s") * nc + lax.axis_index("c")
    for ci in range(16):
        g0 = wid * 512 + ci * 32
        pltpu.sync_copy(ginds_hbm.at[pl.ds(g0, 32)], idx_v)
        pltpu.async_copy(featT_hbm.at[idx_v], rows_v, sem).wait()
        pltpu.sync_copy(rows_v, out_hbm.at[pl.ds(g0, 32)])


def _sc_gather(featT, ginds):
    B, HW, C = featT.shape
    mesh = plsc.VectorSubcoreMesh(core_axis_name="c", subcore_axis_name="s")
    kern = functools.partial(
        pl.kernel,
        out_type=jax.ShapeDtypeStruct((B * 1024, C), jnp.float32),
        scratch_types=[
            pltpu.VMEM((32,), jnp.int32),
            pltpu.VMEM((32, C), jnp.float32),
            pltpu.SemaphoreType.DMA,
        ],
        mesh=mesh,
    )(_gather_body)
    out = kern(featT.reshape(B * HW, C), ginds.reshape(B * 1024))
    return out.reshape(B, 1024, C)


def kernel(features, k):
    B, C, H, W = features.shape
    featT = jnp.transpose(features.reshape(B, C, H * W), (0, 2, 1))
    sim = _sim_map(featT)
    koff = (jnp.asarray(k, jnp.int32) - 1024).reshape(1, 1)
    vals, px, py, ginds = _nms_topk(sim, koff)
    pos = jnp.stack([px, py], axis=-1)
    point_feats = _sc_gather(featT, ginds)
    return point_feats, pos, vals


# speedup vs baseline: 2.2160x; 1.0642x over previous
import functools

import jax
import jax.numpy as jnp
from jax import lax
from jax.experimental import pallas as pl
from jax.experimental.pallas import tpu as pltpu
from jax.experimental.pallas import tpu_sc as plsc


def _red6(x):
    c = [x[:, i * 128:(i + 1) * 128] for i in range(6)]
    r01 = jnp.sum(c[0] + c[1], axis=1, keepdims=True)
    r23 = jnp.sum(c[2] + c[3], axis=1, keepdims=True)
    r45 = jnp.sum(c[4] + c[5], axis=1, keepdims=True)
    return (r01 + r23) + r45


def _sim_body(feat_ref, hup_ref, hdn_ref, out_ref):
    t = pl.program_id(1)
    ft = feat_ref[0]
    hup = hup_ref[0]
    hdn = hdn_ref[0]
    i = lax.broadcasted_iota(jnp.int32, (1024, 1), 0)
    x = jnp.bitwise_and(i, 63)
    y = t * 16 + jnp.right_shift(i, 6)
    eps = jnp.float32(1e-4)

    down = jnp.concatenate([hup[-1:], ft[:-1]], axis=0)
    up = jnp.concatenate([ft[1:], hdn[:1]], axis=0)
    down64 = jnp.concatenate([hup, ft[:-64]], axis=0)
    up64 = jnp.concatenate([ft[64:], hdn], axis=0)

    n_t = _red6(ft * ft)
    n_hup = _red6(hup * hup)
    n_hdn = _red6(hdn * hdn)
    n_full = jnp.concatenate([n_hup, n_t, n_hdn], axis=0)
    sq_full = jnp.sqrt(n_full)
    sq_dn = sq_full[63:1087]
    sq_up = sq_full[65:1089]
    sq_dn64 = sq_full[0:1024]
    sq_up64 = sq_full[128:1152]

    num_h = _red6(down * up)
    num_v = _red6(down64 * up64)
    h = num_h / (sq_dn * sq_up)
    v = num_v / (sq_dn64 * sq_up64)

    f3 = ft.reshape(16, 64, 768)
    r0 = f3[:, 0, :]
    r1 = f3[:, 1, :]
    r62 = f3[:, 62, :]
    r63 = f3[:, 63, :]
    num_x0 = _red6((r0 * r1) * eps)
    e0 = eps * r0
    d1_x0 = _red6(e0 * e0)
    num_x63 = _red6((r62 * r63) * eps)
    e63 = eps * r63
    d2_x63 = _red6(e63 * e63)
    sqn3 = sq_full[64:1088].reshape(16, 64, 1)
    h_x0 = num_x0 / (jnp.sqrt(d1_x0) * sqn3[:, 1, :])
    h_x63 = num_x63 / (sqn3[:, 62, :] * jnp.sqrt(d2_x63))
    h3 = h.reshape(16, 64, 1)
    h3 = jnp.where(lax.broadcasted_iota(jnp.int32, (16, 64, 1), 1) == 0,
                   h_x0[:, None, :], h3)
    h3 = jnp.where(lax.broadcasted_iota(jnp.int32, (16, 64, 1), 1) == 63,
                   h_x63[:, None, :], h3)
    h = h3.reshape(1024, 1)

    s0 = ft[:64]
    s0n = up64[:64]
    num_y0 = _red6((s0n * s0) * eps)
    ey0 = eps * s0
    d2_y0 = _red6(ey0 * ey0)
    v_y0 = num_y0 / (sq_up64[:64] * jnp.sqrt(d2_y0))
    s63 = ft[960:]
    s63p = down64[960:]
    num_y63 = _red6((s63p * s63) * eps)
    ey63 = eps * s63
    d1_y63 = _red6(ey63 * ey63)
    v_y63 = num_y63 / (jnp.sqrt(d1_y63) * sq_dn64[960:])
    zfill = jnp.zeros((960, 1), jnp.float32)
    v = jnp.where(y == 0, jnp.concatenate([v_y0, zfill], axis=0), v)
    v = jnp.where(y == 63, jnp.concatenate([zfill, v_y63], axis=0), v)

    out_ref[0] = (h + v) * jnp.float32(0.5)


def _sim_map(featT):
    B, HW, C = featT.shape
    out = pl.pallas_call(
        _sim_body,
        grid=(B, 4),
        in_specs=[
            pl.BlockSpec((1, 1024, C), lambda b, t: (b, t, 0)),
            pl.BlockSpec((1, 64, C), lambda b, t: (b, jnp.maximum(16 * t - 1, 0), 0)),
            pl.BlockSpec((1, 64, C), lambda b, t: (b, jnp.minimum(16 * t + 16, 63), 0)),
        ],
        out_specs=pl.BlockSpec((1, 1024, 1), lambda b, t: (b, t, 0)),
        out_shape=jax.ShapeDtypeStruct((B, HW, 1), jnp.float32),
    )(featT, featT, featT)
    return out[..., 0]


def _topk_body(sim_ref, koff_ref, vals_ref, px_ref, py_ref, ginds_ref):
    sim = sim_ref[...]
    B, N = sim.shape
    lane = lax.broadcasted_iota(jnp.int32, (B, N), 1)
    xcol = jnp.bitwise_and(lane, 63)
    inf = jnp.float32(jnp.inf)
    one = jnp.float32(1.0)

    sl = jnp.concatenate([sim[:, 1:], sim[:, :1]], axis=1)
    sr = jnp.concatenate([sim[:, -1:], sim[:, :-1]], axis=1)
    nb_r = jnp.where(xcol == 63, inf, sl)
    nb_l = jnp.where(xcol == 0, inf, sr)
    min_h = jnp.minimum(jnp.minimum(nb_l, sim), nb_r)
    su = jnp.concatenate([sim[:, 64:], sim[:, :64]], axis=1)
    sd = jnp.concatenate([sim[:, -64:], sim[:, :-64]], axis=1)
    nb_d = jnp.where(lane >= N - 64, inf, su)
    nb_u = jnp.where(lane < 64, inf, sd)
    min_v = jnp.minimum(jnp.minimum(nb_u, sim), nb_d)

    minima = (sim == min_h) | (sim == min_v)
    sel = jnp.where(minima, one - sim, jnp.float32(0.0))
    score = one - (one - sel)

    val = score
    idx = lane
    for kk in [2 ** e for e in range(1, 13)]:
        j = kk // 2
        while j >= 1:
            rl = jnp.concatenate([val[:, j:], val[:, :j]], axis=1)
            rr = jnp.concatenate([val[:, -j:], val[:, :-j]], axis=1)
            il = jnp.concatenate([idx[:, j:], idx[:, :j]], axis=1)
            ir = jnp.concatenate([idx[:, -j:], idx[:, :-j]], axis=1)
            is_low = (lane & j) == 0
            ov = jnp.where(is_low, rl, rr)
            oi = jnp.where(is_low, il, ir)
            desc = (lane & kk) == 0
            self_greater = (val > ov) | ((val == ov) & (idx < oi))
            keep = self_greater == (desc == is_low)
            val = jnp.where(keep, val, ov)
            idx = jnp.where(keep, idx, oi)
            j //= 2

    koff = koff_ref[0, 0]
    vals_ref[...] = val[:, :1024]
    inds = idx[:, :1024] + koff
    py_ref[...] = jnp.floor_divide(inds, 64)
    px_ref[...] = jnp.remainder(inds, 64)
    gi = jnp.where(inds < 0, inds + 4096, inds)
    gi = jnp.clip(gi, 0, 4095)
    boff = lax.broadcasted_iota(jnp.int32, gi.shape, 0) * 4096
    ginds_ref[...] = gi + boff


def _nms_topk(sim, koff, interpret=False):
    B, N = sim.shape
    return pl.pallas_call(
        _topk_body,
        in_specs=[
            pl.BlockSpec((B, N), lambda: (0, 0)),
            pl.BlockSpec(memory_space=pltpu.SMEM),
        ],
        out_shape=(
            jax.ShapeDtypeStruct((B, 1024), jnp.float32),
            jax.ShapeDtypeStruct((B, 1024), jnp.int32),
            jax.ShapeDtypeStruct((B, 1024), jnp.int32),
            jax.ShapeDtypeStruct((B, 1024), jnp.int32),
        ),
        interpret=interpret,
    )(sim, koff)


def _gather_body(featT_hbm, ginds_hbm, out_hbm, idx_v, rows_v,
                 sg0, sg1, so0, so1):
    nc = 2
    wid = lax.axis_index("s") * nc + lax.axis_index("c")
    sg = [sg0, sg1]
    so = [so0, so1]
    gathers = [None, None]
    outs = [None, None]
    for ci in range(16):
        bi = ci & 1
        g0 = wid * 512 + ci * 32
        if outs[bi] is not None:
            outs[bi].wait()
            outs[bi] = None
        pltpu.sync_copy(ginds_hbm.at[pl.ds(g0, 32)], idx_v.at[bi])
        gathers[bi] = (pltpu.async_copy(
            featT_hbm.at[idx_v.at[bi]], rows_v.at[bi], sg[bi]), g0)
        pbi = 1 - bi
        if gathers[pbi] is not None:
            cp, pg0 = gathers[pbi]
            cp.wait()
            gathers[pbi] = None
            outs[pbi] = pltpu.async_copy(
                rows_v.at[pbi], out_hbm.at[pl.ds(pg0, 32)], so[pbi])
    for bi in range(2):
        if gathers[bi] is not None:
            cp, pg0 = gathers[bi]
            cp.wait()
            outs[bi] = pltpu.async_copy(
                rows_v.at[bi], out_hbm.at[pl.ds(pg0, 32)], so[bi])
    for bi in range(2):
        if outs[bi] is not None:
            outs[bi].wait()


def _sc_gather(featT, ginds):
    B, HW, C = featT.shape
    mesh = plsc.VectorSubcoreMesh(core_axis_name="c", subcore_axis_name="s")
    kern = functools.partial(
        pl.kernel,
        out_type=jax.ShapeDtypeStruct((B * 1024, C), jnp.float32),
        scratch_types=[
            pltpu.VMEM((2, 32), jnp.int32),
            pltpu.VMEM((2, 32, C), jnp.float32),
            pltpu.SemaphoreType.DMA,
            pltpu.SemaphoreType.DMA,
            pltpu.SemaphoreType.DMA,
            pltpu.SemaphoreType.DMA,
        ],
        mesh=mesh,
    )(_gather_body)
    out = kern(featT.reshape(B * HW, C), ginds.reshape(B * 1024))
    return out.reshape(B, 1024, C)


def kernel(features, k):
    B, C, H, W = features.shape
    featT = jnp.transpose(features.reshape(B, C, H * W), (0, 2, 1))
    sim = _sim_map(featT)
    koff = (jnp.asarray(k, jnp.int32) - 1024).reshape(1, 1)
    vals, px, py, ginds = _nms_topk(sim, koff)
    pos = jnp.stack([px, py], axis=-1)
    point_feats = _sc_gather(featT, ginds)
    return point_feats, pos, vals


# speedup vs baseline: 2.2312x; 1.0069x over previous
import functools

import jax
import jax.numpy as jnp
from jax import lax
from jax.experimental import pallas as pl
from jax.experimental.pallas import tpu as pltpu
from jax.experimental.pallas import tpu_sc as plsc


def _red6(x):
    c = [x[:, i * 128:(i + 1) * 128] for i in range(6)]
    r01 = jnp.sum(c[0] + c[1], axis=1, keepdims=True)
    r23 = jnp.sum(c[2] + c[3], axis=1, keepdims=True)
    r45 = jnp.sum(c[4] + c[5], axis=1, keepdims=True)
    return (r01 + r23) + r45


def _sim_body(feat_ref, hup_ref, hdn_ref, out_ref):
    t = pl.program_id(1)
    ft = feat_ref[0]
    hup = hup_ref[0]
    hdn = hdn_ref[0]
    i = lax.broadcasted_iota(jnp.int32, (1024, 1), 0)
    x = jnp.bitwise_and(i, 63)
    y = t * 16 + jnp.right_shift(i, 6)
    eps = jnp.float32(1e-4)

    down = jnp.concatenate([hup[-1:], ft[:-1]], axis=0)
    up = jnp.concatenate([ft[1:], hdn[:1]], axis=0)
    down64 = jnp.concatenate([hup, ft[:-64]], axis=0)
    up64 = jnp.concatenate([ft[64:], hdn], axis=0)

    n_t = _red6(ft * ft)
    n_hup = _red6(hup * hup)
    n_hdn = _red6(hdn * hdn)
    n_full = jnp.concatenate([n_hup, n_t, n_hdn], axis=0)
    sq_full = jnp.sqrt(n_full)
    sq_dn = sq_full[63:1087]
    sq_up = sq_full[65:1089]
    sq_dn64 = sq_full[0:1024]
    sq_up64 = sq_full[128:1152]

    num_h = _red6(down * up)
    num_v = _red6(down64 * up64)
    h = num_h / (sq_dn * sq_up)
    v = num_v / (sq_dn64 * sq_up64)

    f3 = ft.reshape(16, 64, 768)
    r0 = f3[:, 0, :]
    r1 = f3[:, 1, :]
    r62 = f3[:, 62, :]
    r63 = f3[:, 63, :]
    num_x0 = _red6((r0 * r1) * eps)
    e0 = eps * r0
    d1_x0 = _red6(e0 * e0)
    num_x63 = _red6((r62 * r63) * eps)
    e63 = eps * r63
    d2_x63 = _red6(e63 * e63)
    sqn3 = sq_full[64:1088].reshape(16, 64, 1)
    h_x0 = num_x0 / (jnp.sqrt(d1_x0) * sqn3[:, 1, :])
    h_x63 = num_x63 / (sqn3[:, 62, :] * jnp.sqrt(d2_x63))
    h3 = h.reshape(16, 64, 1)
    h3 = jnp.where(lax.broadcasted_iota(jnp.int32, (16, 64, 1), 1) == 0,
                   h_x0[:, None, :], h3)
    h3 = jnp.where(lax.broadcasted_iota(jnp.int32, (16, 64, 1), 1) == 63,
                   h_x63[:, None, :], h3)
    h = h3.reshape(1024, 1)

    s0 = ft[:64]
    s0n = up64[:64]
    num_y0 = _red6((s0n * s0) * eps)
    ey0 = eps * s0
    d2_y0 = _red6(ey0 * ey0)
    v_y0 = num_y0 / (sq_up64[:64] * jnp.sqrt(d2_y0))
    s63 = ft[960:]
    s63p = down64[960:]
    num_y63 = _red6((s63p * s63) * eps)
    ey63 = eps * s63
    d1_y63 = _red6(ey63 * ey63)
    v_y63 = num_y63 / (jnp.sqrt(d1_y63) * sq_dn64[960:])
    zfill = jnp.zeros((960, 1), jnp.float32)
    v = jnp.where(y == 0, jnp.concatenate([v_y0, zfill], axis=0), v)
    v = jnp.where(y == 63, jnp.concatenate([zfill, v_y63], axis=0), v)

    out_ref[0] = (h + v) * jnp.float32(0.5)


def _sim_map(featT):
    B, HW, C = featT.shape
    out = pl.pallas_call(
        _sim_body,
        grid=(B, 4),
        in_specs=[
            pl.BlockSpec((1, 1024, C), lambda b, t: (b, t, 0)),
            pl.BlockSpec((1, 64, C), lambda b, t: (b, jnp.maximum(16 * t - 1, 0), 0)),
            pl.BlockSpec((1, 64, C), lambda b, t: (b, jnp.minimum(16 * t + 16, 63), 0)),
        ],
        out_specs=pl.BlockSpec((1, 1024, 1), lambda b, t: (b, t, 0)),
        out_shape=jax.ShapeDtypeStruct((B, HW, 1), jnp.float32),
    )(featT, featT, featT)
    return out[..., 0]


def _topk_body(sim_ref, koff_ref, vals_ref, px_ref, py_ref, ginds_ref):
    sim = sim_ref[...]
    B, N = sim.shape
    lane = lax.broadcasted_iota(jnp.int32, (B, N), 1)
    xcol = jnp.bitwise_and(lane, 63)
    inf = jnp.float32(jnp.inf)
    one = jnp.float32(1.0)

    sl = jnp.concatenate([sim[:, 1:], sim[:, :1]], axis=1)
    sr = jnp.concatenate([sim[:, -1:], sim[:, :-1]], axis=1)
    nb_r = jnp.where(xcol == 63, inf, sl)
    nb_l = jnp.where(xcol == 0, inf, sr)
    min_h = jnp.minimum(jnp.minimum(nb_l, sim), nb_r)
    su = jnp.concatenate([sim[:, 64:], sim[:, :64]], axis=1)
    sd = jnp.concatenate([sim[:, -64:], sim[:, :-64]], axis=1)
    nb_d = jnp.where(lane >= N - 64, inf, su)
    nb_u = jnp.where(lane < 64, inf, sd)
    min_v = jnp.minimum(jnp.minimum(nb_u, sim), nb_d)

    minima = (sim == min_h) | (sim == min_v)
    sel = jnp.where(minima, one - sim, jnp.float32(0.0))
    score = one - (one - sel)

    val = score
    idx = lane
    for kk in [2 ** e for e in range(1, 13)]:
        j = kk // 2
        while j >= 1:
            rl = jnp.concatenate([val[:, j:], val[:, :j]], axis=1)
            rr = jnp.concatenate([val[:, -j:], val[:, :-j]], axis=1)
            il = jnp.concatenate([idx[:, j:], idx[:, :j]], axis=1)
            ir = jnp.concatenate([idx[:, -j:], idx[:, :-j]], axis=1)
            is_low = (lane & j) == 0
            ov = jnp.where(is_low, rl, rr)
            oi = jnp.where(is_low, il, ir)
            desc = (lane & kk) == 0
            self_greater = (val > ov) | ((val == ov) & (idx < oi))
            keep = self_greater == (desc == is_low)
            val = jnp.where(keep, val, ov)
            idx = jnp.where(keep, idx, oi)
            j //= 2

    koff = koff_ref[0, 0]
    vals_ref[...] = val[:, :1024]
    inds = idx[:, :1024] + koff
    py_ref[...] = jnp.floor_divide(inds, 64)
    px_ref[...] = jnp.remainder(inds, 64)
    gi = jnp.where(inds < 0, inds + 4096, inds)
    gi = jnp.clip(gi, 0, 4095)
    boff = lax.broadcasted_iota(jnp.int32, gi.shape, 0) * 4096
    ginds_ref[...] = gi + boff


def _nms_topk(sim, koff, interpret=False):
    B, N = sim.shape
    return pl.pallas_call(
        _topk_body,
        in_specs=[
            pl.BlockSpec((B, N), lambda: (0, 0)),
            pl.BlockSpec(memory_space=pltpu.SMEM),
        ],
        out_shape=(
            jax.ShapeDtypeStruct((B, 1024), jnp.float32),
            jax.ShapeDtypeStruct((B, 1024), jnp.int32),
            jax.ShapeDtypeStruct((B, 1024), jnp.int32),
            jax.ShapeDtypeStruct((B, 1024), jnp.int32),
        ),
        interpret=interpret,
    )(sim, koff)


def _gather_body(featT_hbm, ginds_hbm, out_hbm, idx_v, rows_v,
                 sg0, sg1, so0, so1):
    nc = 2
    wid = lax.axis_index("s") * nc + lax.axis_index("c")
    sg = [sg0, sg1]
    so = [so0, so1]
    gathers = [None, None]
    outs = [None, None]
    for ci in range(8):
        bi = ci & 1
        g0 = wid * 512 + ci * 64
        if outs[bi] is not None:
            outs[bi].wait()
            outs[bi] = None
        pltpu.sync_copy(ginds_hbm.at[pl.ds(g0, 64)], idx_v.at[bi])
        gathers[bi] = (pltpu.async_copy(
            featT_hbm.at[idx_v.at[bi]], rows_v.at[bi], sg[bi]), g0)
        pbi = 1 - bi
        if gathers[pbi] is not None:
            cp, pg0 = gathers[pbi]
            cp.wait()
            gathers[pbi] = None
            outs[pbi] = pltpu.async_copy(
                rows_v.at[pbi], out_hbm.at[pl.ds(pg0, 64)], so[pbi])
    for bi in range(2):
        if gathers[bi] is not None:
            cp, pg0 = gathers[bi]
            cp.wait()
            outs[bi] = pltpu.async_copy(
                rows_v.at[bi], out_hbm.at[pl.ds(pg0, 64)], so[bi])
    for bi in range(2):
        if outs[bi] is not None:
            outs[bi].wait()


def _sc_gather(featT, ginds):
    B, HW, C = featT.shape
    mesh = plsc.VectorSubcoreMesh(core_axis_name="c", subcore_axis_name="s")
    kern = functools.partial(
        pl.kernel,
        out_type=jax.ShapeDtypeStruct((B * 1024, C), jnp.float32),
        scratch_types=[
            pltpu.VMEM((2, 64), jnp.int32),
            pltpu.VMEM((2, 64, C), jnp.float32),
            pltpu.SemaphoreType.DMA,
            pltpu.SemaphoreType.DMA,
            pltpu.SemaphoreType.DMA,
            pltpu.SemaphoreType.DMA,
        ],
        mesh=mesh,
    )(_gather_body)
    out = kern(featT.reshape(B * HW, C), ginds.reshape(B * 1024))
    return out.reshape(B, 1024, C)


def kernel(features, k):
    B, C, H, W = features.shape
    featT = jnp.transpose(features.reshape(B, C, H * W), (0, 2, 1))
    sim = _sim_map(featT)
    koff = (jnp.asarray(k, jnp.int32) - 1024).reshape(1, 1)
    vals, px, py, ginds = _nms_topk(sim, koff)
    pos = jnp.stack([px, py], axis=-1)
    point_feats = _sc_gather(featT, ginds)
    return point_feats, pos, vals
